# baseline (speedup 1.0000x reference)
import functools

import jax
import jax.numpy as jnp
from jax import lax
from jax.experimental import pallas as pl
from jax.experimental.pallas import tpu as pltpu
from jax.experimental.pallas import tpu_sc as plsc

B = 16384
D = 32
H = 128
PK = H // D
NC = 2
NS = 16
NW = NC * NS
BPW = B // NW


def _sc_gather_body(idx_div, t0, t1, t2, t3, out,
                    idx0, idx1, idx2, idx3, rows, sem):
    wid = lax.axis_index("s") * NC + lax.axis_index("c")
    base = wid * BPW
    tables = (t0, t1, t2, t3)
    idx_bufs = (idx0, idx1, idx2, idx3)
    for t in range(4):
        pltpu.sync_copy(idx_div.at[t, pl.ds(base, BPW)], idx_bufs[t])
    for t in range(4):
        pltpu.async_copy(tables[t].at[idx_bufs[t]], rows, sem).wait()
        pltpu.sync_copy(rows, out.at[t, pl.ds(base, BPW)])


def _sc_gather(idx_div, t0, t1, t2, t3):
    mesh = plsc.VectorSubcoreMesh(core_axis_name="c", subcore_axis_name="s")
    scratch = (
        [pltpu.VMEM((BPW,), jnp.int32) for _ in range(4)]
        + [pltpu.VMEM((BPW, H), jnp.float32)]
        + [pltpu.SemaphoreType.DMA]
    )
    k = pl.kernel(
        _sc_gather_body,
        out_type=jax.ShapeDtypeStruct((4, B, H), jnp.float32),
        mesh=mesh,
        scratch_types=scratch,
    )
    return k(idx_div, t0, t1, t2, t3)


BM = 2048


def _mlp_body(g_ref, q_ref, w1_ref, b1_ref, w2_ref, b2_ref, out_ref):
    lane = lax.broadcasted_iota(jnp.int32, (BM, H), 1)
    h = b1_ref[0, :][None, :]
    for t in range(4):
        lo = q_ref[t] * D
        mask = (lane >= lo) & (lane < lo + D)
        xm = jnp.where(mask, g_ref[t], 0.0)
        h = h + jnp.dot(xm, w1_ref[t], preferred_element_type=jnp.float32)
    h = jnp.maximum(h, 0.0)
    out_ref[...] = (
        jnp.sum(h * w2_ref[0, :][None, :], axis=1, keepdims=True) + b2_ref[0, 0]
    )


def _mlp(g, q, W1, b1, W2, b2):
    w1e = jnp.tile(W1.reshape(4, D, H), (1, PK, 1))
    w2_row = W2.reshape(1, H)
    b1_row = b1.reshape(1, H)
    b2_s = b2.reshape(1, 1)
    out = pl.pallas_call(
        _mlp_body,
        grid=(B // BM,),
        in_specs=[
            pl.BlockSpec((4, BM, H), lambda i: (0, i, 0)),
            pl.BlockSpec((4, BM, 1), lambda i: (0, i, 0)),
            pl.BlockSpec((4, H, H), lambda i: (0, 0, 0)),
            pl.BlockSpec((1, H), lambda i: (0, 0)),
            pl.BlockSpec((1, H), lambda i: (0, 0)),
            pl.BlockSpec((1, 1), lambda i: (0, 0)),
        ],
        out_specs=pl.BlockSpec((BM, 1), lambda i: (i, 0)),
        out_shape=jax.ShapeDtypeStruct((B, 1), jnp.float32),
    )(g, q, w1e, b1_row, w2_row, b2_s)
    return out[:, 0]


def kernel(user, item, language, category,
           user_emb, item_emb, language_emb, category_emb,
           W1, b1, W2, b2):
    idx = jnp.stack([
        user.astype(jnp.int32), item.astype(jnp.int32),
        language.astype(jnp.int32), category.astype(jnp.int32)])
    idx_div = idx // PK
    q = (idx % PK).reshape(4, B, 1)
    g = _sc_gather(
        idx_div,
        user_emb.reshape(-1, H), item_emb.reshape(-1, H),
        language_emb.reshape(-1, H), category_emb.reshape(-1, H))
    return _mlp(g, q, W1, b1, W2, b2)
